# baseline (speedup 1.0000x reference)
import jax
import jax.numpy as jnp
from jax.experimental import pallas as pl
from jax.experimental.pallas import tpu as pltpu

D_MODEL = 512
NUM_HEADS = 8
HEAD_DIM = 64
MAX_CACHE = 4096
T = 16
TQ = 17
V = 256
EPS = 1e-5
NEG = float(jnp.finfo(jnp.float32).min)


def _ln(x, g, b):
    m = jnp.mean(x, axis=-1, keepdims=True)
    v = jnp.mean((x - m) ** 2, axis=-1, keepdims=True)
    return (x - m) * jax.lax.rsqrt(v + EPS) * g + b


HG = 4
JL = NUM_HEADS // HG - 1


def _stream_kernel(sc_ref,
                   vis_ref, ckT_ref, cvT_ref, cls_ref, cur_ref,
                   qn_g, qn_b, vn_g, vn_b, q_w, q_b,
                   o_w, o_b, fn_g, fn_b, f1_w, f1_b, f2_w, f2_b,
                   cn_g, cn_b, k_w, k_b, v_w, v_b,
                   okT_ref, ovT_ref, ncls_ref, ntok_ref, kapp_ref, vapp_ref,
                   ctx_scr, qh_scr, kvis_scr, vvis_scr, kcur_scr, vcur_scr):
    b = pl.program_id(0)
    j = pl.program_id(1)
    vl = sc_ref[b, 0]
    okT_ref[...] = ckT_ref[...]
    ovT_ref[...] = cvT_ref[...]

    @pl.when(j == 0)
    def _():
        x = jnp.concatenate([cls_ref[0], cur_ref[0]], axis=0)
        qi = _ln(x, qn_g[0], qn_b[0])
        q = (qi @ q_w[...] + q_b[0]) * (HEAD_DIM ** -0.5)
        vis = _ln(vis_ref[0], vn_g[0], vn_b[0])
        kv = vis @ k_w[...] + k_b[0]
        vv = vis @ v_w[...] + v_b[0]
        cu = qi[1:, :]
        kc = cu @ k_w[...] + k_b[0]
        vc = cu @ v_w[...] + v_b[0]
        for hh in range(NUM_HEADS):
            sl = slice(hh * HEAD_DIM, (hh + 1) * HEAD_DIM)
            qh_scr[hh] = q[:, sl]
            kvis_scr[hh] = kv[:, sl]
            vvis_scr[hh] = vv[:, sl]
            kcur_scr[hh] = kc[:, sl]
            vcur_scr[hh] = vc[:, sl]

    kidx = jax.lax.broadcasted_iota(jnp.int32, (1, MAX_CACHE), 1)
    cache_dead = kidx >= vl
    mrow = jnp.stack([sc_ref[b, 2 + t] for t in range(T)]).reshape(1, T)
    cur_live = mrow > 0
    dims_nt = (((1,), (1,)), ((), ()))
    for h in range(HG):
        hid = j * HG + h
        q = qh_scr[hid]
        kT = ckT_ref[0, h]
        vT = cvT_ref[0, h]
        s_vis = jax.lax.dot_general(q, kvis_scr[hid], dims_nt)
        s_cache = jax.lax.dot_general(q, kT, (((1,), (0,)), ((), ())))
        s_cur = jax.lax.dot_general(q, kcur_scr[hid], dims_nt)

        s_cache = jnp.where(cache_dead, NEG, s_cache)
        s_cur = jnp.where(cur_live, s_cur, NEG)

        m = jnp.maximum(
            jnp.maximum(jnp.max(s_vis, axis=-1, keepdims=True),
                        jnp.max(s_cur, axis=-1, keepdims=True)),
            jnp.max(s_cache, axis=-1, keepdims=True))
        e_vis = jnp.exp(s_vis - m)
        e_cache = jnp.exp(s_cache - m)
        e_cur = jnp.exp(s_cur - m)
        l = (jnp.sum(e_vis, axis=-1, keepdims=True)
             + jnp.sum(e_cache, axis=-1, keepdims=True)
             + jnp.sum(e_cur, axis=-1, keepdims=True))
        acc = (jnp.dot(e_vis, vvis_scr[hid])
               + jax.lax.dot_general(e_cache, vT, dims_nt)
               + jnp.dot(e_cur, vcur_scr[hid]))
        ctx_scr[hid] = acc / l

    @pl.when(j == JL)
    def _():
        ctx = jnp.concatenate([ctx_scr[hh] for hh in range(NUM_HEADS)], axis=1)
        att = ctx @ o_w[...] + o_b[0]
        x = jnp.concatenate([cls_ref[0], cur_ref[0]], axis=0) + att
        h1 = _ln(x, fn_g[0], fn_b[0]) @ f1_w[...] + f1_b[0]
        g = h1 * 0.5 * (1.0 + jax.lax.erf(h1 * (2.0 ** -0.5)))
        x = x + g @ f2_w[...] + f2_b[0]
        ncls_ref[0] = jnp.where(sc_ref[b, 1] > 0, x[0:1, :], cls_ref[0])
        mcol = jnp.stack([sc_ref[b, 2 + t] for t in range(T)]).reshape(T, 1)
        ntok = x[1:, :] * mcol.astype(jnp.float32)
        ntok_ref[0] = ntok
        ct = _ln(ntok, cn_g[0], cn_b[0])
        kapp_ref[0] = ct @ k_w[...] + k_b[0]
        vapp_ref[0] = ct @ v_w[...] + v_b[0]


SB = 4
SW = 256


def _scatter_kernel(sc_ref,
                    ckT_ref, cvT_ref, kapp_ref, vapp_ref,
                    nkT_ref, nvT_ref,
                    win_k, win_v, sems):
    step = pl.program_id(0)
    starts = []
    for i in range(SB):
        b = step * SB + i
        start = pl.multiple_of((sc_ref[b, 0] // 128) * 128, 128)
        starts.append(start)
        pltpu.make_async_copy(
            nkT_ref.at[b, :, :, pl.ds(start, SW)], win_k.at[i], sems.at[i, 0]
        ).start()
        pltpu.make_async_copy(
            nvT_ref.at[b, :, :, pl.ds(start, SW)], win_v.at[i], sems.at[i, 1]
        ).start()
    dims_tl = (((0,), (0,)), ((), ()))
    for i in range(SB):
        b = step * SB + i
        vl = sc_ref[b, 0]
        act = sc_ref[b, 1]
        cum = 0
        dest = []
        for t in range(T):
            mt = sc_ref[b, 2 + t]
            cum = cum + mt
            dest.append(jnp.where((mt > 0) & (act > 0), vl + cum - 1, -1))
        dest_col = jnp.stack(dest).reshape(T, 1)
        lanes = starts[i] + jax.lax.broadcasted_iota(jnp.int32, (1, SW), 1)
        tm = jnp.where(dest_col == lanes, 1.0, 0.0)
        wcol = jnp.sum(tm, axis=0, keepdims=True) > 0.0
        pltpu.make_async_copy(win_k.at[i], win_k.at[i], sems.at[i, 0]).wait()
        pltpu.make_async_copy(win_v.at[i], win_v.at[i], sems.at[i, 1]).wait()
        for h in range(NUM_HEADS):
            sl = slice(h * HEAD_DIM, (h + 1) * HEAD_DIM)
            newk = jax.lax.dot_general(kapp_ref[i][:, sl], tm, dims_tl)
            newv = jax.lax.dot_general(vapp_ref[i][:, sl], tm, dims_tl)
            win_k[i, h] = jnp.where(wcol, newk, win_k[i, h])
            win_v[i, h] = jnp.where(wcol, newv, win_v[i, h])
        pltpu.make_async_copy(
            win_k.at[i], nkT_ref.at[b, :, :, pl.ds(starts[i], SW)], sems.at[i, 0]
        ).start()
        pltpu.make_async_copy(
            win_v.at[i], nvT_ref.at[b, :, :, pl.ds(starts[i], SW)], sems.at[i, 1]
        ).start()
    for i in range(SB):
        b = step * SB + i
        pltpu.make_async_copy(
            win_k.at[i], nkT_ref.at[b, :, :, pl.ds(starts[i], SW)], sems.at[i, 0]
        ).wait()
        pltpu.make_async_copy(
            win_v.at[i], nvT_ref.at[b, :, :, pl.ds(starts[i], SW)], sems.at[i, 1]
        ).wait()


def kernel(prev_cls_state, current_tokens, visual_tokens, cache_key, cache_value,
           params, token_valid_mask, sample_active, cache_valid_len):
    p = params
    B = prev_cls_state.shape[0]
    f32 = jnp.float32
    cls3 = prev_cls_state.reshape(B, 1, D_MODEL)
    row = lambda name: p[name].reshape(1, -1)
    mask_i = token_valid_mask.astype(jnp.int32)
    scal = jnp.concatenate(
        [cache_valid_len.astype(jnp.int32)[:, None],
         sample_active.astype(jnp.int32)[:, None], mask_i], axis=1)
    vl = cache_valid_len.astype(jnp.int32)

    ckT = cache_key.transpose(0, 1, 3, 2)
    cvT = cache_value.transpose(0, 1, 3, 2)
    bhT = pl.BlockSpec((1, HG, HEAD_DIM, MAX_CACHE), lambda b, h, *_: (b, h, 0, 0))
    bD = lambda s: pl.BlockSpec((1, s, D_MODEL), lambda b, h, *_: (b, 0, 0))
    f2 = lambda *shape: pl.BlockSpec(shape, lambda b, h, *_: (0,) * len(shape))
    sD = lambda s: jax.ShapeDtypeStruct((B, s, D_MODEL), f32)
    hscr = lambda s: pltpu.VMEM((NUM_HEADS, s, HEAD_DIM), f32)
    cacheT_sds = jax.ShapeDtypeStruct((B, NUM_HEADS, HEAD_DIM, MAX_CACHE), f32)
    new_keyT, new_valueT, ncls, ntok, kapp, vapp = pl.pallas_call(
        _stream_kernel,
        grid_spec=pltpu.PrefetchScalarGridSpec(
            num_scalar_prefetch=1,
            grid=(B, NUM_HEADS // HG),
            in_specs=[
                bD(V), bhT, bhT, bD(1), bD(T),
                f2(1, D_MODEL), f2(1, D_MODEL), f2(1, D_MODEL), f2(1, D_MODEL),
                f2(D_MODEL, D_MODEL), f2(1, D_MODEL),
                f2(D_MODEL, D_MODEL), f2(1, D_MODEL),
                f2(1, D_MODEL), f2(1, D_MODEL),
                f2(D_MODEL, 4 * D_MODEL), f2(1, 4 * D_MODEL),
                f2(4 * D_MODEL, D_MODEL), f2(1, D_MODEL),
                f2(1, D_MODEL), f2(1, D_MODEL),
                f2(D_MODEL, D_MODEL), f2(1, D_MODEL),
                f2(D_MODEL, D_MODEL), f2(1, D_MODEL),
            ],
            out_specs=[bhT, bhT, bD(1), bD(T), bD(T), bD(T)],
            scratch_shapes=[hscr(TQ), hscr(TQ), hscr(V), hscr(V),
                            hscr(T), hscr(T)],
        ),
        out_shape=[cacheT_sds, cacheT_sds, sD(1), sD(T), sD(T), sD(T)],
        compiler_params=pltpu.CompilerParams(
            dimension_semantics=("parallel", "arbitrary"),
            vmem_limit_bytes=57 * 1024 * 1024),
        name="stream",
    )(scal,
      visual_tokens, ckT, cvT, cls3, current_tokens,
      row('qn_g'), row('qn_b'), row('vn_g'), row('vn_b'),
      p['q_w'], row('q_b'),
      p['o_w'], row('o_b'), row('fn_g'), row('fn_b'),
      p['f1_w'], row('f1_b'), p['f2_w'], row('f2_b'),
      row('cn_g'), row('cn_b'), p['k_w'], row('k_b'), p['v_w'], row('v_b'))

    new_keyT, new_valueT = pl.pallas_call(
        _scatter_kernel,
        grid_spec=pltpu.PrefetchScalarGridSpec(
            num_scalar_prefetch=1,
            grid=(B // SB,),
            in_specs=[
                pl.BlockSpec(memory_space=pl.ANY),
                pl.BlockSpec(memory_space=pl.ANY),
                pl.BlockSpec((SB, T, D_MODEL), lambda s, sc_s: (s, 0, 0)),
                pl.BlockSpec((SB, T, D_MODEL), lambda s, sc_s: (s, 0, 0)),
            ],
            out_specs=[pl.BlockSpec(memory_space=pl.ANY),
                       pl.BlockSpec(memory_space=pl.ANY)],
            scratch_shapes=[
                pltpu.VMEM((SB, NUM_HEADS, HEAD_DIM, SW), f32),
                pltpu.VMEM((SB, NUM_HEADS, HEAD_DIM, SW), f32),
                pltpu.SemaphoreType.DMA((SB, 2)),
            ],
        ),
        out_shape=[cacheT_sds, cacheT_sds],
        input_output_aliases={1: 0, 2: 1},
        compiler_params=pltpu.CompilerParams(
            dimension_semantics=("parallel",)),
        name="scatter",
    )(scal, new_keyT, new_valueT, kapp, vapp)
    new_key = new_keyT.transpose(0, 1, 3, 2)
    new_value = new_valueT.transpose(0, 1, 3, 2)

    next_cls = ncls.reshape(B, D_MODEL)
    new_valid_len = jnp.where(sample_active, vl + mask_i.sum(axis=1), vl)
    return next_cls, ntok, new_key, new_value, new_valid_len


# speedup vs baseline: 1.0159x; 1.0159x over previous
import jax
import jax.numpy as jnp
from jax.experimental import pallas as pl
from jax.experimental.pallas import tpu as pltpu

D_MODEL = 512
NUM_HEADS = 8
HEAD_DIM = 64
MAX_CACHE = 4096
T = 16
TQ = 17
V = 256
EPS = 1e-5
NEG = float(jnp.finfo(jnp.float32).min)


def _ln(x, g, b):
    m = jnp.mean(x, axis=-1, keepdims=True)
    v = jnp.mean((x - m) ** 2, axis=-1, keepdims=True)
    return (x - m) * jax.lax.rsqrt(v + EPS) * g + b


HG = 4
JL = NUM_HEADS // HG - 1


def _stream_kernel(sc_ref,
                   vis_ref, ckT_ref, cvT_ref, cls_ref, cur_ref,
                   qn_g, qn_b, vn_g, vn_b, q_w, q_b,
                   o_w, o_b, fn_g, fn_b, f1_w, f1_b, f2_w, f2_b,
                   cn_g, cn_b, k_w, k_b, v_w, v_b,
                   okT_ref, ovT_ref, ncls_ref, ntok_ref, kapp_ref, vapp_ref,
                   ctx_scr, qh_scr, kvis_scr, vvis_scr, kcur_scr, vcur_scr):
    b = pl.program_id(0)
    j = pl.program_id(1)
    vl = sc_ref[b, 0]
    cls_row = cls_ref[pl.ds(b, 1), :]
    okT_ref[...] = ckT_ref[...]
    ovT_ref[...] = cvT_ref[...]

    @pl.when(j == 0)
    def _():
        x = jnp.concatenate([cls_row, cur_ref[0]], axis=0)
        qi = _ln(x, qn_g[0], qn_b[0])
        q = (qi @ q_w[...] + q_b[0]) * (HEAD_DIM ** -0.5)
        vis = _ln(vis_ref[0], vn_g[0], vn_b[0])
        kv = vis @ k_w[...] + k_b[0]
        vv = vis @ v_w[...] + v_b[0]
        cu = qi[1:, :]
        kc = cu @ k_w[...] + k_b[0]
        vc = cu @ v_w[...] + v_b[0]
        for hh in range(NUM_HEADS):
            sl = slice(hh * HEAD_DIM, (hh + 1) * HEAD_DIM)
            qh_scr[hh] = q[:, sl]
            kvis_scr[hh] = kv[:, sl]
            vvis_scr[hh] = vv[:, sl]
            kcur_scr[hh] = kc[:, sl]
            vcur_scr[hh] = vc[:, sl]

    kidx = jax.lax.broadcasted_iota(jnp.int32, (1, MAX_CACHE), 1)
    cache_dead = kidx >= vl
    mrow = jnp.stack([sc_ref[b, 2 + t] for t in range(T)]).reshape(1, T)
    cur_live = mrow > 0
    dims_nt = (((1,), (1,)), ((), ()))
    for h in range(HG):
        hid = j * HG + h
        q = qh_scr[hid]
        kT = ckT_ref[0, h]
        vT = cvT_ref[0, h]
        s_vis = jax.lax.dot_general(q, kvis_scr[hid], dims_nt)
        s_cache = jax.lax.dot_general(q, kT, (((1,), (0,)), ((), ())))
        s_cur = jax.lax.dot_general(q, kcur_scr[hid], dims_nt)

        s_cache = jnp.where(cache_dead, NEG, s_cache)
        s_cur = jnp.where(cur_live, s_cur, NEG)

        m = jnp.maximum(
            jnp.maximum(jnp.max(s_vis, axis=-1, keepdims=True),
                        jnp.max(s_cur, axis=-1, keepdims=True)),
            jnp.max(s_cache, axis=-1, keepdims=True))
        e_vis = jnp.exp(s_vis - m)
        e_cache = jnp.exp(s_cache - m)
        e_cur = jnp.exp(s_cur - m)
        l = (jnp.sum(e_vis, axis=-1, keepdims=True)
             + jnp.sum(e_cache, axis=-1, keepdims=True)
             + jnp.sum(e_cur, axis=-1, keepdims=True))
        acc = (jnp.dot(e_vis, vvis_scr[hid])
               + jax.lax.dot_general(e_cache, vT, dims_nt)
               + jnp.dot(e_cur, vcur_scr[hid]))
        ctx_scr[hid] = acc / l

    @pl.when(j == JL)
    def _():
        ctx = jnp.concatenate([ctx_scr[hh] for hh in range(NUM_HEADS)], axis=1)
        att = ctx @ o_w[...] + o_b[0]
        x = jnp.concatenate([cls_row, cur_ref[0]], axis=0) + att
        h1 = _ln(x, fn_g[0], fn_b[0]) @ f1_w[...] + f1_b[0]
        g = h1 * 0.5 * (1.0 + jax.lax.erf(h1 * (2.0 ** -0.5)))
        x = x + g @ f2_w[...] + f2_b[0]
        ncls_ref[0] = jnp.where(sc_ref[b, 1] > 0, x[0:1, :], cls_row)
        mcol = jnp.stack([sc_ref[b, 2 + t] for t in range(T)]).reshape(T, 1)
        ntok = x[1:, :] * mcol.astype(jnp.float32)
        ntok_ref[0] = ntok
        ct = _ln(ntok, cn_g[0], cn_b[0])
        kapp_ref[0] = ct @ k_w[...] + k_b[0]
        vapp_ref[0] = ct @ v_w[...] + v_b[0]


SB = 4
SW = 256


def _scatter_kernel(sc_ref,
                    ckT_ref, cvT_ref, kapp_ref, vapp_ref,
                    nkT_ref, nvT_ref,
                    win_k, win_v, sems):
    step = pl.program_id(0)
    starts = []
    for i in range(SB):
        b = step * SB + i
        start = pl.multiple_of((sc_ref[b, 0] // 128) * 128, 128)
        starts.append(start)
        pltpu.make_async_copy(
            nkT_ref.at[b, :, :, pl.ds(start, SW)], win_k.at[i], sems.at[i, 0]
        ).start()
        pltpu.make_async_copy(
            nvT_ref.at[b, :, :, pl.ds(start, SW)], win_v.at[i], sems.at[i, 1]
        ).start()
    dims_tl = (((0,), (0,)), ((), ()))
    for i in range(SB):
        b = step * SB + i
        vl = sc_ref[b, 0]
        act = sc_ref[b, 1]
        cum = 0
        dest = []
        for t in range(T):
            mt = sc_ref[b, 2 + t]
            cum = cum + mt
            dest.append(jnp.where((mt > 0) & (act > 0), vl + cum - 1, -1))
        dest_col = jnp.stack(dest).reshape(T, 1)
        lanes = starts[i] + jax.lax.broadcasted_iota(jnp.int32, (1, SW), 1)
        tm = jnp.where(dest_col == lanes, 1.0, 0.0)
        wcol = jnp.sum(tm, axis=0, keepdims=True) > 0.0
        pltpu.make_async_copy(win_k.at[i], win_k.at[i], sems.at[i, 0]).wait()
        pltpu.make_async_copy(win_v.at[i], win_v.at[i], sems.at[i, 1]).wait()
        for h in range(NUM_HEADS):
            sl = slice(h * HEAD_DIM, (h + 1) * HEAD_DIM)
            newk = jax.lax.dot_general(kapp_ref[i][:, sl], tm, dims_tl)
            newv = jax.lax.dot_general(vapp_ref[i][:, sl], tm, dims_tl)
            win_k[i, h] = jnp.where(wcol, newk, win_k[i, h])
            win_v[i, h] = jnp.where(wcol, newv, win_v[i, h])
        pltpu.make_async_copy(
            win_k.at[i], nkT_ref.at[b, :, :, pl.ds(starts[i], SW)], sems.at[i, 0]
        ).start()
        pltpu.make_async_copy(
            win_v.at[i], nvT_ref.at[b, :, :, pl.ds(starts[i], SW)], sems.at[i, 1]
        ).start()
    for i in range(SB):
        b = step * SB + i
        pltpu.make_async_copy(
            win_k.at[i], nkT_ref.at[b, :, :, pl.ds(starts[i], SW)], sems.at[i, 0]
        ).wait()
        pltpu.make_async_copy(
            win_v.at[i], nvT_ref.at[b, :, :, pl.ds(starts[i], SW)], sems.at[i, 1]
        ).wait()


def kernel(prev_cls_state, current_tokens, visual_tokens, cache_key, cache_value,
           params, token_valid_mask, sample_active, cache_valid_len):
    p = params
    B = prev_cls_state.shape[0]
    f32 = jnp.float32
    row = lambda name: p[name].reshape(1, -1)
    mask_i = token_valid_mask.astype(jnp.int32)
    scal = jnp.concatenate(
        [cache_valid_len.astype(jnp.int32)[:, None],
         sample_active.astype(jnp.int32)[:, None], mask_i], axis=1)
    vl = cache_valid_len.astype(jnp.int32)

    ckT = cache_key.transpose(0, 1, 3, 2)
    cvT = cache_value.transpose(0, 1, 3, 2)
    bhT = pl.BlockSpec((1, HG, HEAD_DIM, MAX_CACHE), lambda b, h, *_: (b, h, 0, 0))
    bD = lambda s: pl.BlockSpec((1, s, D_MODEL), lambda b, h, *_: (b, 0, 0))
    f2 = lambda *shape: pl.BlockSpec(shape, lambda b, h, *_: (0,) * len(shape))
    sD = lambda s: jax.ShapeDtypeStruct((B, s, D_MODEL), f32)
    hscr = lambda s: pltpu.VMEM((NUM_HEADS, s, HEAD_DIM), f32)
    cacheT_sds = jax.ShapeDtypeStruct((B, NUM_HEADS, HEAD_DIM, MAX_CACHE), f32)
    new_keyT, new_valueT, ncls, ntok, kapp, vapp = pl.pallas_call(
        _stream_kernel,
        grid_spec=pltpu.PrefetchScalarGridSpec(
            num_scalar_prefetch=1,
            grid=(B, NUM_HEADS // HG),
            in_specs=[
                bD(V), bhT, bhT, f2(B, D_MODEL), bD(T),
                f2(1, D_MODEL), f2(1, D_MODEL), f2(1, D_MODEL), f2(1, D_MODEL),
                f2(D_MODEL, D_MODEL), f2(1, D_MODEL),
                f2(D_MODEL, D_MODEL), f2(1, D_MODEL),
                f2(1, D_MODEL), f2(1, D_MODEL),
                f2(D_MODEL, 4 * D_MODEL), f2(1, 4 * D_MODEL),
                f2(4 * D_MODEL, D_MODEL), f2(1, D_MODEL),
                f2(1, D_MODEL), f2(1, D_MODEL),
                f2(D_MODEL, D_MODEL), f2(1, D_MODEL),
                f2(D_MODEL, D_MODEL), f2(1, D_MODEL),
            ],
            out_specs=[bhT, bhT, bD(1), bD(T), bD(T), bD(T)],
            scratch_shapes=[hscr(TQ), hscr(TQ), hscr(V), hscr(V),
                            hscr(T), hscr(T)],
        ),
        out_shape=[cacheT_sds, cacheT_sds, sD(1), sD(T), sD(T), sD(T)],
        compiler_params=pltpu.CompilerParams(
            dimension_semantics=("parallel", "arbitrary"),
            vmem_limit_bytes=57 * 1024 * 1024),
        name="stream",
    )(scal,
      visual_tokens, ckT, cvT, prev_cls_state, current_tokens,
      row('qn_g'), row('qn_b'), row('vn_g'), row('vn_b'),
      p['q_w'], row('q_b'),
      p['o_w'], row('o_b'), row('fn_g'), row('fn_b'),
      p['f1_w'], row('f1_b'), p['f2_w'], row('f2_b'),
      row('cn_g'), row('cn_b'), p['k_w'], row('k_b'), p['v_w'], row('v_b'))

    new_keyT, new_valueT = pl.pallas_call(
        _scatter_kernel,
        grid_spec=pltpu.PrefetchScalarGridSpec(
            num_scalar_prefetch=1,
            grid=(B // SB,),
            in_specs=[
                pl.BlockSpec(memory_space=pl.ANY),
                pl.BlockSpec(memory_space=pl.ANY),
                pl.BlockSpec((SB, T, D_MODEL), lambda s, sc_s: (s, 0, 0)),
                pl.BlockSpec((SB, T, D_MODEL), lambda s, sc_s: (s, 0, 0)),
            ],
            out_specs=[pl.BlockSpec(memory_space=pl.ANY),
                       pl.BlockSpec(memory_space=pl.ANY)],
            scratch_shapes=[
                pltpu.VMEM((SB, NUM_HEADS, HEAD_DIM, SW), f32),
                pltpu.VMEM((SB, NUM_HEADS, HEAD_DIM, SW), f32),
                pltpu.SemaphoreType.DMA((SB, 2)),
            ],
        ),
        out_shape=[cacheT_sds, cacheT_sds],
        input_output_aliases={1: 0, 2: 1},
        compiler_params=pltpu.CompilerParams(
            dimension_semantics=("parallel",)),
        name="scatter",
    )(scal, new_keyT, new_valueT, kapp, vapp)
    new_key = new_keyT.transpose(0, 1, 3, 2)
    new_value = new_valueT.transpose(0, 1, 3, 2)

    next_cls = ncls.reshape(B, D_MODEL)
    new_valid_len = jnp.where(sample_active, vl + mask_i.sum(axis=1), vl)
    return next_cls, ntok, new_key, new_value, new_valid_len
